# Initial kernel scaffold; baseline (speedup 1.0000x reference)
import functools

import jax
import jax.numpy as jnp
from jax import lax
from jax.experimental import pallas as pl
from jax.experimental.pallas import tpu as pltpu
from jax.experimental.pallas import tpu_sc as plsc

N_NODES = 10000
N_EDGES = 320000
D = 128
NR = 10240
DA = 144
DUMMY = N_NODES
K = 128
NTILES = 32
EPAD = 323584
EPT = EPAD // NTILES
NCHUNK = EPT // K
RPT = NR // 16
BLK = 1024


def _row_mask_col(i):
    rid = i * BLK + lax.broadcasted_iota(jnp.int32, (BLK, 1), 0)
    return jnp.where(rid < N_NODES, 1.0, 0.0).astype(jnp.float32)


def _t1_body(x_ref, w_ref, o_ref):
    i = pl.program_id(0)
    proj = jnp.dot(x_ref[...], w_ref[...], preferred_element_type=jnp.float32)
    pad = jnp.zeros((BLK, DA - D - 1), jnp.float32)
    o_ref[...] = jnp.concatenate([proj, _row_mask_col(i), pad], axis=1)


def _tidx_body(ei_ref, row_ref, col_ref):
    rowv = ei_ref[0:1, :]
    colv = ei_ref[1:2, :]
    cmin = jnp.min(colv)
    fill = jnp.full((1, EPAD - N_EDGES), DUMMY, jnp.int32)
    row_ref[0:1, 0:N_EDGES] = rowv
    row_ref[0:1, N_EDGES:EPAD] = fill
    col_ref[0:1, 0:N_EDGES] = colv - cmin
    col_ref[0:1, N_EDGES:EPAD] = fill


def _count_col(s):
    onehot = (lax.broadcasted_iota(jnp.int32, (1, DA), 1) == D).astype(jnp.float32)
    cnt = jnp.sum(s * onehot, axis=1, keepdims=True)
    return jnp.maximum(cnt, 1.0)


def _t2_body(p_ref, w_ref, o_ref):
    i = pl.program_id(0)
    s = p_ref[0] + p_ref[1]
    feat = s[:, :D] * (1.0 / _count_col(s))
    proj = jnp.dot(feat, w_ref[...], preferred_element_type=jnp.float32)
    pad = jnp.zeros((BLK, DA - D - 1), jnp.float32)
    o_ref[...] = jnp.concatenate([proj, _row_mask_col(i), pad], axis=1)


def _t3_body(p_ref, xa_ref, wu_ref, b_ref, o_ref):
    s = p_ref[0] + p_ref[1]
    nagg = s[:, :D] * (1.0 / _count_col(s))
    xp = xa_ref[:, :D]
    acc = (jnp.dot(xp, wu_ref[:D], preferred_element_type=jnp.float32)
           + jnp.dot(nagg, wu_ref[D:], preferred_element_type=jnp.float32)
           + b_ref[...])
    o_ref[...] = jnp.maximum(acc, 0.0)


def _sc_stage(table, gidx, sidx, zeros_nr):
    mesh = plsc.VectorSubcoreMesh(core_axis_name="c", subcore_axis_name="s")

    @functools.partial(
        pl.kernel, mesh=mesh,
        out_type=jax.ShapeDtypeStruct((2, NR, DA), jnp.float32),
        scratch_types=[
            pltpu.VMEM((K,), jnp.int32),
            pltpu.VMEM((K,), jnp.int32),
            pltpu.VMEM((K, DA), jnp.float32),
            pltpu.VMEM_SHARED((NR, DA), jnp.float32),
            pltpu.SemaphoreType.DMA,
        ],
    )
    def run(table_h, gidx_h, sidx_h, zeros_h, out_h, gi_v, si_v, rows_v, acc, sem):
        c = lax.axis_index("c")
        s = lax.axis_index("s")
        wid = s * 2 + c
        pltpu.sync_copy(zeros_h.at[pl.ds(s * RPT, RPT)], acc.at[pl.ds(s * RPT, RPT)])
        plsc.subcore_barrier()
        base = wid * EPT

        def body(k, carry):
            off = base + k * K
            pltpu.sync_copy(gidx_h.at[pl.ds(off, K)], gi_v)
            pltpu.sync_copy(sidx_h.at[pl.ds(off, K)], si_v)
            pltpu.async_copy(table_h.at[gi_v], rows_v, sem).wait()
            pltpu.sync_copy(rows_v, acc.at[si_v], add=True)
            return carry

        lax.fori_loop(0, NCHUNK, body, 0)
        plsc.subcore_barrier()
        pltpu.sync_copy(acc.at[pl.ds(s * RPT, RPT)],
                        out_h.at[c, pl.ds(s * RPT, RPT)])

    return run(table, gidx, sidx, zeros_nr)


def _t1(x_pad, W_v):
    return pl.pallas_call(
        _t1_body,
        grid=(NR // BLK,),
        in_specs=[pl.BlockSpec((BLK, D), lambda i: (i, 0)),
                  pl.BlockSpec((D, D), lambda i: (0, 0))],
        out_specs=pl.BlockSpec((BLK, DA), lambda i: (i, 0)),
        out_shape=jax.ShapeDtypeStruct((NR, DA), jnp.float32),
    )(x_pad, W_v)


def _tidx(edge_index):
    return pl.pallas_call(
        _tidx_body,
        in_specs=[pl.BlockSpec((2, N_EDGES), lambda: (0, 0))],
        out_specs=[pl.BlockSpec((1, EPAD), lambda: (0, 0)),
                   pl.BlockSpec((1, EPAD), lambda: (0, 0))],
        out_shape=[jax.ShapeDtypeStruct((1, EPAD), jnp.int32),
                   jax.ShapeDtypeStruct((1, EPAD), jnp.int32)],
    )(edge_index)


def _t2(pA, W_e):
    return pl.pallas_call(
        _t2_body,
        grid=(NR // BLK,),
        in_specs=[pl.BlockSpec((2, BLK, DA), lambda i: (0, i, 0)),
                  pl.BlockSpec((D, D), lambda i: (0, 0))],
        out_specs=pl.BlockSpec((BLK, DA), lambda i: (i, 0)),
        out_shape=jax.ShapeDtypeStruct((NR, DA), jnp.float32),
    )(pA, W_e)


def _t3(pB, x_aug, W_u, b_u):
    return pl.pallas_call(
        _t3_body,
        grid=(NR // BLK,),
        in_specs=[pl.BlockSpec((2, BLK, DA), lambda i: (0, i, 0)),
                  pl.BlockSpec((BLK, DA), lambda i: (i, 0)),
                  pl.BlockSpec((2 * D, D), lambda i: (0, 0)),
                  pl.BlockSpec((1, D), lambda i: (0, 0))],
        out_specs=pl.BlockSpec((BLK, D), lambda i: (i, 0)),
        out_shape=jax.ShapeDtypeStruct((NR, D), jnp.float32),
    )(pB, x_aug, W_u, b_u)


def kernel(x, edge_index, W_v, W_e, W_u, b_u):
    x_pad = jnp.zeros((NR, D), jnp.float32).at[:N_NODES].set(x)
    x_aug = _t1(x_pad, W_v)
    row_pad, col_pad = _tidx(edge_index)
    row_pad = row_pad.reshape(EPAD)
    col_pad = col_pad.reshape(EPAD)
    zeros_nr = jnp.zeros((NR, DA), jnp.float32)
    pA = _sc_stage(x_aug, row_pad, col_pad, zeros_nr)
    e_aug = _t2(pA, W_e)
    pB = _sc_stage(e_aug, col_pad, row_pad, zeros_nr)
    outp = _t3(pB, x_aug, W_u, b_u.reshape(1, D))
    return outp[:N_NODES]


# speedup vs baseline: 3.6773x; 3.6773x over previous
import functools

import jax
import jax.numpy as jnp
from jax import lax
from jax.experimental import pallas as pl
from jax.experimental.pallas import tpu as pltpu
from jax.experimental.pallas import tpu_sc as plsc

N_NODES = 10000
N_EDGES = 320000
D = 128
NR = 10240
DUMMY = N_NODES
K = 128
NTILES = 32
EPAD = 323584
EPT = EPAD // NTILES
NCHUNK = EPT // K
RPT = NR // 16
BLK = 1024


def _t1_body(x_ref, w_ref, o_ref):
    o_ref[...] = jnp.dot(x_ref[...], w_ref[...], preferred_element_type=jnp.float32)


def _tidx_body(ei_ref, row_ref, col_ref):
    rowv = ei_ref[0:1, :]
    colv = ei_ref[1:2, :]
    cmin = jnp.min(colv)
    fill = jnp.full((1, EPAD - N_EDGES), DUMMY, jnp.int32)
    row_ref[0:1, 0:N_EDGES] = rowv
    row_ref[0:1, N_EDGES:EPAD] = fill
    col_ref[0:1, 0:N_EDGES] = colv - cmin
    col_ref[0:1, N_EDGES:EPAD] = fill


def _inv_cnt(cnt_ref):
    cnt = jnp.sum(cnt_ref[...], axis=0, keepdims=True)
    inv = 1.0 / jnp.maximum(cnt, 1.0)
    return jnp.transpose(inv, (1, 0))


def _t2_body(p_ref, cnt_ref, w_ref, o_ref):
    s = p_ref[0] + p_ref[1]
    feat = s * _inv_cnt(cnt_ref)
    o_ref[...] = jnp.dot(feat, w_ref[...], preferred_element_type=jnp.float32)


def _t3_body(p_ref, cnt_ref, xp_ref, wu_ref, b_ref, o_ref):
    s = p_ref[0] + p_ref[1]
    nagg = s * _inv_cnt(cnt_ref)
    acc = (jnp.dot(xp_ref[...], wu_ref[:D], preferred_element_type=jnp.float32)
           + jnp.dot(nagg, wu_ref[D:], preferred_element_type=jnp.float32)
           + b_ref[...])
    o_ref[...] = jnp.maximum(acc, 0.0)


def _sc_stage(table, gidx, sidx, zeros_nr):
    mesh = plsc.VectorSubcoreMesh(core_axis_name="c", subcore_axis_name="s")

    @functools.partial(
        pl.kernel, mesh=mesh,
        compiler_params=pltpu.CompilerParams(needs_layout_passes=False),
        out_type=(jax.ShapeDtypeStruct((2, NR, D), jnp.float32),
                  jax.ShapeDtypeStruct((NTILES, NR), jnp.float32)),
        scratch_types=[
            pltpu.VMEM((K,), jnp.int32),
            pltpu.VMEM((K,), jnp.int32),
            pltpu.VMEM((K, D), jnp.float32),
            pltpu.VMEM((NR,), jnp.float32),
            pltpu.VMEM_SHARED((NR, D), jnp.float32),
            pltpu.SemaphoreType.DMA,
        ],
    )
    def run(table_h, gidx_h, sidx_h, zeros_h, out_h, hist_h,
            gi_v, si_v, rows_v, hist_v, acc, sem):
        c = lax.axis_index("c")
        s = lax.axis_index("s")
        wid = s * 2 + c
        pltpu.sync_copy(zeros_h.at[pl.ds(s * RPT, RPT)], acc.at[pl.ds(s * RPT, RPT)])

        zeros16 = jnp.zeros((16,), jnp.float32)

        def zbody(i, carry):
            hist_v[pl.ds(i * 16, 16)] = zeros16
            return carry

        lax.fori_loop(0, NR // 16, zbody, 0)
        plsc.subcore_barrier()

        base = wid * EPT
        ones16 = jnp.full((16,), 1.0, jnp.float32)

        def body(k, carry):
            off = base + k * K
            pltpu.sync_copy(gidx_h.at[pl.ds(off, K)], gi_v)
            pltpu.sync_copy(sidx_h.at[pl.ds(off, K)], si_v)
            pltpu.async_copy(table_h.at[gi_v], rows_v, sem).wait()
            pltpu.sync_copy(rows_v, acc.at[si_v], add=True)

            def hbody(j, hcarry):
                idx16 = si_v[pl.ds(j * 16, 16)]
                plsc.addupdate_scatter(hist_v, [idx16], ones16)
                return hcarry

            lax.fori_loop(0, K // 16, hbody, 0)
            return carry

        lax.fori_loop(0, NCHUNK, body, 0)
        plsc.subcore_barrier()
        pltpu.sync_copy(acc.at[pl.ds(s * RPT, RPT)],
                        out_h.at[c, pl.ds(s * RPT, RPT)])
        pltpu.sync_copy(hist_v, hist_h.at[wid])

    return run(table, gidx, sidx, zeros_nr)


def _t1(x_pad, W_v):
    return pl.pallas_call(
        _t1_body,
        grid=(NR // BLK,),
        in_specs=[pl.BlockSpec((BLK, D), lambda i: (i, 0)),
                  pl.BlockSpec((D, D), lambda i: (0, 0))],
        out_specs=pl.BlockSpec((BLK, D), lambda i: (i, 0)),
        out_shape=jax.ShapeDtypeStruct((NR, D), jnp.float32),
    )(x_pad, W_v)


def _tidx(edge_index):
    return pl.pallas_call(
        _tidx_body,
        in_specs=[pl.BlockSpec((2, N_EDGES), lambda: (0, 0))],
        out_specs=[pl.BlockSpec((1, EPAD), lambda: (0, 0)),
                   pl.BlockSpec((1, EPAD), lambda: (0, 0))],
        out_shape=[jax.ShapeDtypeStruct((1, EPAD), jnp.int32),
                   jax.ShapeDtypeStruct((1, EPAD), jnp.int32)],
    )(edge_index)


def _t2(pA, cntA, W_e):
    return pl.pallas_call(
        _t2_body,
        grid=(NR // BLK,),
        in_specs=[pl.BlockSpec((2, BLK, D), lambda i: (0, i, 0)),
                  pl.BlockSpec((NTILES, BLK), lambda i: (0, i)),
                  pl.BlockSpec((D, D), lambda i: (0, 0))],
        out_specs=pl.BlockSpec((BLK, D), lambda i: (i, 0)),
        out_shape=jax.ShapeDtypeStruct((NR, D), jnp.float32),
    )(pA, cntA, W_e)


def _t3(pB, cntB, x_proj, W_u, b_u):
    return pl.pallas_call(
        _t3_body,
        grid=(NR // BLK,),
        in_specs=[pl.BlockSpec((2, BLK, D), lambda i: (0, i, 0)),
                  pl.BlockSpec((NTILES, BLK), lambda i: (0, i)),
                  pl.BlockSpec((BLK, D), lambda i: (i, 0)),
                  pl.BlockSpec((2 * D, D), lambda i: (0, 0)),
                  pl.BlockSpec((1, D), lambda i: (0, 0))],
        out_specs=pl.BlockSpec((BLK, D), lambda i: (i, 0)),
        out_shape=jax.ShapeDtypeStruct((NR, D), jnp.float32),
    )(pB, cntB, x_proj, W_u, b_u)


def kernel(x, edge_index, W_v, W_e, W_u, b_u):
    x_pad = jnp.zeros((NR, D), jnp.float32).at[:N_NODES].set(x)
    x_proj = _t1(x_pad, W_v)
    row_pad, col_pad = _tidx(edge_index)
    row_pad = row_pad.reshape(EPAD)
    col_pad = col_pad.reshape(EPAD)
    zeros_nr = jnp.zeros((NR, D), jnp.float32)
    pA, cntA = _sc_stage(x_proj, row_pad, col_pad, zeros_nr)
    e_proj = _t2(pA, cntA, W_e)
    pB, cntB = _sc_stage(e_proj, col_pad, row_pad, zeros_nr)
    outp = _t3(pB, cntB, x_proj, W_u, b_u.reshape(1, D))
    return outp[:N_NODES]
